# baseline (speedup 1.0000x reference)
import functools

import jax
import jax.numpy as jnp
from jax.experimental import pallas as pl
from jax.experimental.pallas import tpu as pltpu

N = 50000
E = 800000
NG = 50
NF = 128
HC = 256
TH = 32
EMB_DIM = HC - TH
NZ = 85
NTAG = 3
NROWS = NZ + NTAG

EDGE_BLK = 2048
ATOM_BLK = 2048


def _silu(x):
    return x * (1.0 / (1.0 + jnp.exp(-x)))


def _edge_body(rp_ref, ea_ref, we1_ref, be1_ref, we12_ref, be12_ref,
               we2_ref, be2_ref, out_ref):
    rp = rp_ref[...]
    ea = ea_ref[...]
    u = _silu(jnp.dot(rp, we1_ref[...], preferred_element_type=jnp.float32)
              + be1_ref[...])
    v = _silu(jnp.dot(ea, we12_ref[...], preferred_element_type=jnp.float32)
              + be12_ref[...])
    pre = (jnp.dot(u, we2_ref[:NF // 2, :], preferred_element_type=jnp.float32)
           + jnp.dot(v, we2_ref[NF // 2:, :], preferred_element_type=jnp.float32)
           + be2_ref[...])
    out_ref[...] = _silu(pre)


def _atom_body(z_ref, tag_ref, table_ref, wl_ref, bl_ref, wl2_ref, bl2_ref,
               out_ref):
    zb = z_ref[...]
    tb = tag_ref[...]
    cols = jax.lax.broadcasted_iota(jnp.int32, (zb.shape[0], NROWS), 1)
    oh = ((zb == cols) | ((tb + NZ) == cols)).astype(jnp.float32)
    h0 = jnp.dot(oh, table_ref[...], preferred_element_type=jnp.float32)
    h1 = _silu(jnp.dot(h0, wl_ref[...], preferred_element_type=jnp.float32)
               + bl_ref[...])
    out_ref[...] = _silu(jnp.dot(h1, wl2_ref[...],
                                 preferred_element_type=jnp.float32)
                         + bl2_ref[...])


def _full(shape):
    return pl.BlockSpec(shape, lambda i: (0,) * len(shape))


def kernel(z, rel_pos, edge_attr, tag, emb_table, tag_table, W_lin, b_lin,
           W_lin2, b_lin2, W_e1, b_e1, W_e12, b_e12, W_e2, b_e2):
    e = pl.pallas_call(
        _edge_body,
        grid=(pl.cdiv(E, EDGE_BLK),),
        in_specs=[
            pl.BlockSpec((EDGE_BLK, 3), lambda i: (i, 0)),
            pl.BlockSpec((EDGE_BLK, NG), lambda i: (i, 0)),
            _full((3, NF // 2)),
            _full((1, NF // 2)),
            _full((NG, NF - NF // 2)),
            _full((1, NF - NF // 2)),
            _full((NF, NF)),
            _full((1, NF)),
        ],
        out_specs=pl.BlockSpec((EDGE_BLK, NF), lambda i: (i, 0)),
        out_shape=jax.ShapeDtypeStruct((E, NF), jnp.float32),
    )(rel_pos, edge_attr, W_e1, b_e1.reshape(1, -1), W_e12,
      b_e12.reshape(1, -1), W_e2, b_e2.reshape(1, -1))

    table = jnp.zeros((NROWS, HC), dtype=jnp.float32)
    table = table.at[:NZ, :EMB_DIM].set(emb_table)
    table = table.at[NZ:, EMB_DIM:].set(tag_table)

    h = pl.pallas_call(
        _atom_body,
        grid=(pl.cdiv(N, ATOM_BLK),),
        in_specs=[
            pl.BlockSpec((ATOM_BLK, 1), lambda i: (i, 0)),
            pl.BlockSpec((ATOM_BLK, 1), lambda i: (i, 0)),
            _full((NROWS, HC)),
            _full((HC, HC)),
            _full((1, HC)),
            _full((HC, HC)),
            _full((1, HC)),
        ],
        out_specs=pl.BlockSpec((ATOM_BLK, HC), lambda i: (i, 0)),
        out_shape=jax.ShapeDtypeStruct((N, HC), jnp.float32),
    )(z.reshape(-1, 1), tag.reshape(-1, 1), table, W_lin,
      b_lin.reshape(1, -1), W_lin2, b_lin2.reshape(1, -1))

    return (h, e)


# speedup vs baseline: 1.1898x; 1.1898x over previous
import functools

import jax
import jax.numpy as jnp
from jax.experimental import pallas as pl
from jax.experimental.pallas import tpu as pltpu

N = 50000
E = 800000
NG = 50
NF = 128
HC = 256
TH = 32
EMB_DIM = HC - TH
NZ = 85
NTAG = 3
NROWS = NZ + NTAG

EDGE_BLK = 8192
ATOM_BLK = 8192


def _silu(x):
    return x * (1.0 / (1.0 + jnp.exp(-x)))


def _edge_body(rp_ref, ea_ref, we1_ref, be1_ref, we12_ref, be12_ref,
               we2_ref, be2_ref, out_ref):
    rp = rp_ref[...]
    ea = ea_ref[...]
    u = _silu(jnp.dot(rp, we1_ref[...], preferred_element_type=jnp.float32)
              + be1_ref[...])
    v = _silu(jnp.dot(ea, we12_ref[...], preferred_element_type=jnp.float32)
              + be12_ref[...])
    pre = (jnp.dot(u, we2_ref[:NF // 2, :], preferred_element_type=jnp.float32)
           + jnp.dot(v, we2_ref[NF // 2:, :], preferred_element_type=jnp.float32)
           + be2_ref[...])
    out_ref[...] = _silu(pre)


def _atom_body(z_ref, tag_ref, table_ref, wl_ref, bl_ref, wl2_ref, bl2_ref,
               out_ref):
    zb = z_ref[...]
    tb = tag_ref[...]
    cols = jax.lax.broadcasted_iota(jnp.int32, (zb.shape[0], NROWS), 1)
    oh = ((zb == cols) | ((tb + NZ) == cols)).astype(jnp.float32)
    h0 = jnp.dot(oh, table_ref[...], preferred_element_type=jnp.float32)
    h1 = _silu(jnp.dot(h0, wl_ref[...], preferred_element_type=jnp.float32)
               + bl_ref[...])
    out_ref[...] = _silu(jnp.dot(h1, wl2_ref[...],
                                 preferred_element_type=jnp.float32)
                         + bl2_ref[...])


def _full(shape):
    return pl.BlockSpec(shape, lambda i: (0,) * len(shape))


def kernel(z, rel_pos, edge_attr, tag, emb_table, tag_table, W_lin, b_lin,
           W_lin2, b_lin2, W_e1, b_e1, W_e12, b_e12, W_e2, b_e2):
    e = pl.pallas_call(
        _edge_body,
        grid=(pl.cdiv(E, EDGE_BLK),),
        in_specs=[
            pl.BlockSpec((EDGE_BLK, 3), lambda i: (i, 0)),
            pl.BlockSpec((EDGE_BLK, NG), lambda i: (i, 0)),
            _full((3, NF // 2)),
            _full((1, NF // 2)),
            _full((NG, NF - NF // 2)),
            _full((1, NF - NF // 2)),
            _full((NF, NF)),
            _full((1, NF)),
        ],
        out_specs=pl.BlockSpec((EDGE_BLK, NF), lambda i: (i, 0)),
        out_shape=jax.ShapeDtypeStruct((E, NF), jnp.float32),
    )(rel_pos, edge_attr, W_e1, b_e1.reshape(1, -1), W_e12,
      b_e12.reshape(1, -1), W_e2, b_e2.reshape(1, -1))

    table = jnp.zeros((NROWS, HC), dtype=jnp.float32)
    table = table.at[:NZ, :EMB_DIM].set(emb_table)
    table = table.at[NZ:, EMB_DIM:].set(tag_table)

    h = pl.pallas_call(
        _atom_body,
        grid=(pl.cdiv(N, ATOM_BLK),),
        in_specs=[
            pl.BlockSpec((ATOM_BLK, 1), lambda i: (i, 0)),
            pl.BlockSpec((ATOM_BLK, 1), lambda i: (i, 0)),
            _full((NROWS, HC)),
            _full((HC, HC)),
            _full((1, HC)),
            _full((HC, HC)),
            _full((1, HC)),
        ],
        out_specs=pl.BlockSpec((ATOM_BLK, HC), lambda i: (i, 0)),
        out_shape=jax.ShapeDtypeStruct((N, HC), jnp.float32),
    )(z.reshape(-1, 1), tag.reshape(-1, 1), table, W_lin,
      b_lin.reshape(1, -1), W_lin2, b_lin2.reshape(1, -1))

    return (h, e)


# speedup vs baseline: 1.2025x; 1.0107x over previous
import jax
import jax.numpy as jnp
from jax.experimental import pallas as pl
from jax.experimental.pallas import tpu as pltpu

N = 50000
E = 800000
NG = 50
NF = 128
HC = 256
TH = 32
EMB_DIM = HC - TH
NZ = 85
NTAG = 3
NROWS = NZ + NTAG

EDGE_BLK = 8000
ATOM_BLK = 512
NSTEPS = E // EDGE_BLK
ATOM_STEPS = pl.cdiv(N, ATOM_BLK)


def _silu(x):
    return x * (1.0 / (1.0 + jnp.exp(-x)))


def _body(rp_ref, ea_ref, z_ref, tag_ref, table_ref,
          we1_ref, be1_ref, we12_ref, be12_ref, we2_ref, be2_ref,
          wl_ref, bl_ref, wl2_ref, bl2_ref,
          h_ref, e_ref):
    i = pl.program_id(0)

    rp = rp_ref[...]
    ea = ea_ref[...]
    u = _silu(jnp.dot(rp, we1_ref[...], preferred_element_type=jnp.float32)
              + be1_ref[...])
    v = _silu(jnp.dot(ea, we12_ref[...], preferred_element_type=jnp.float32)
              + be12_ref[...])
    pre = (jnp.dot(u, we2_ref[:NF // 2, :], preferred_element_type=jnp.float32)
           + jnp.dot(v, we2_ref[NF // 2:, :], preferred_element_type=jnp.float32)
           + be2_ref[...])
    e_ref[...] = _silu(pre)

    @pl.when(i < ATOM_STEPS)
    def _():
        zb = z_ref[...]
        tb = tag_ref[...]
        cols = jax.lax.broadcasted_iota(jnp.int32, (ATOM_BLK, NROWS), 1)
        oh = ((zb == cols) | ((tb + NZ) == cols)).astype(jnp.float32)
        h0 = jnp.dot(oh, table_ref[...], preferred_element_type=jnp.float32)
        h1 = _silu(jnp.dot(h0, wl_ref[...], preferred_element_type=jnp.float32)
                   + bl_ref[...])
        h_ref[...] = _silu(jnp.dot(h1, wl2_ref[...],
                                   preferred_element_type=jnp.float32)
                           + bl2_ref[...])


def _full(shape):
    return pl.BlockSpec(shape, lambda i: (0,) * len(shape))


def _atom_idx(i):
    c = jnp.minimum(i, ATOM_STEPS - 1)
    return (c, 0)


def kernel(z, rel_pos, edge_attr, tag, emb_table, tag_table, W_lin, b_lin,
           W_lin2, b_lin2, W_e1, b_e1, W_e12, b_e12, W_e2, b_e2):
    table = jnp.zeros((NROWS, HC), dtype=jnp.float32)
    table = table.at[:NZ, :EMB_DIM].set(emb_table)
    table = table.at[NZ:, EMB_DIM:].set(tag_table)

    h, e = pl.pallas_call(
        _body,
        grid=(NSTEPS,),
        in_specs=[
            pl.BlockSpec((EDGE_BLK, 3), lambda i: (i, 0)),
            pl.BlockSpec((EDGE_BLK, NG), lambda i: (i, 0)),
            pl.BlockSpec((ATOM_BLK, 1), _atom_idx),
            pl.BlockSpec((ATOM_BLK, 1), _atom_idx),
            _full((NROWS, HC)),
            _full((3, NF // 2)),
            _full((1, NF // 2)),
            _full((NG, NF - NF // 2)),
            _full((1, NF - NF // 2)),
            _full((NF, NF)),
            _full((1, NF)),
            _full((HC, HC)),
            _full((1, HC)),
            _full((HC, HC)),
            _full((1, HC)),
        ],
        out_specs=[
            pl.BlockSpec((ATOM_BLK, HC), _atom_idx),
            pl.BlockSpec((EDGE_BLK, NF), lambda i: (i, 0)),
        ],
        out_shape=[
            jax.ShapeDtypeStruct((N, HC), jnp.float32),
            jax.ShapeDtypeStruct((E, NF), jnp.float32),
        ],
    )(rel_pos, edge_attr, z.reshape(-1, 1), tag.reshape(-1, 1), table,
      W_e1, b_e1.reshape(1, -1), W_e12, b_e12.reshape(1, -1),
      W_e2, b_e2.reshape(1, -1),
      W_lin, b_lin.reshape(1, -1), W_lin2, b_lin2.reshape(1, -1))

    return (h, e)


# speedup vs baseline: 1.2516x; 1.0408x over previous
import jax
import jax.numpy as jnp
from jax import lax
from jax.experimental import pallas as pl
from jax.experimental.pallas import tpu as pltpu

N = 50000
E = 800000
NG = 50
NF = 128
HC = 256
TH = 32
EMB_DIM = HC - TH
NZ = 85
NTAG = 3
NROWS = NZ + NTAG

EDGE_BLK = 10000
NSTEPS = E // EDGE_BLK
ATOM_BLK = 632
ATOM_STEPS = pl.cdiv(N, ATOM_BLK)


def _silu(x):
    return x * (1.0 / (1.0 + jnp.exp(-x)))


def _body(rp_ref, ea_ref, z_ref, tag_ref, table_ref,
          we1_ref, be1_ref, we12_ref, be12_ref, we2_ref, be2_ref,
          wl_ref, bl_ref, wl2_ref, bl2_ref,
          h_ref, e_ref):
    rp = rp_ref[...]
    ea = ea_ref[...]
    u = _silu(jnp.dot(rp, we1_ref[...], preferred_element_type=jnp.float32)
              + be1_ref[...])
    v = _silu(jnp.dot(ea, we12_ref[...], preferred_element_type=jnp.float32)
              + be12_ref[...])
    pre = (jnp.dot(u, we2_ref[:NF // 2, :], preferred_element_type=jnp.float32)
           + jnp.dot(v, we2_ref[NF // 2:, :], preferred_element_type=jnp.float32)
           + be2_ref[...])
    e_ref[...] = _silu(pre)

    zb = z_ref[...]
    tb = tag_ref[...]
    cols = lax.broadcasted_iota(jnp.int32, (ATOM_BLK, NROWS), 1)
    oh = ((zb == cols) | ((tb + NZ) == cols)).astype(jnp.float32)
    h0 = jnp.dot(oh, table_ref[...], preferred_element_type=jnp.float32)
    h1 = _silu(jnp.dot(h0, wl_ref[...], preferred_element_type=jnp.float32)
               + bl_ref[...])
    h_ref[...] = _silu(jnp.dot(h1, wl2_ref[...],
                               preferred_element_type=jnp.float32)
                       + bl2_ref[...])


def _full(shape):
    return pl.BlockSpec(shape, lambda i: (0,) * len(shape))


def _atom_idx(i):
    return (jnp.minimum(i, ATOM_STEPS - 1), 0)


def kernel(z, rel_pos, edge_attr, tag, emb_table, tag_table, W_lin, b_lin,
           W_lin2, b_lin2, W_e1, b_e1, W_e12, b_e12, W_e2, b_e2):
    table = jnp.zeros((NROWS, HC), dtype=jnp.float32)
    table = table.at[:NZ, :EMB_DIM].set(emb_table)
    table = table.at[NZ:, EMB_DIM:].set(tag_table)

    h, e = pl.pallas_call(
        _body,
        grid=(NSTEPS,),
        in_specs=[
            pl.BlockSpec((EDGE_BLK, 3), lambda i: (i, 0)),
            pl.BlockSpec((EDGE_BLK, NG), lambda i: (i, 0)),
            pl.BlockSpec((ATOM_BLK, 1), _atom_idx),
            pl.BlockSpec((ATOM_BLK, 1), _atom_idx),
            _full((NROWS, HC)),
            _full((3, NF // 2)),
            _full((1, NF // 2)),
            _full((NG, NF - NF // 2)),
            _full((1, NF - NF // 2)),
            _full((NF, NF)),
            _full((1, NF)),
            _full((HC, HC)),
            _full((1, HC)),
            _full((HC, HC)),
            _full((1, HC)),
        ],
        out_specs=[
            pl.BlockSpec((ATOM_BLK, HC), _atom_idx),
            pl.BlockSpec((EDGE_BLK, NF), lambda i: (i, 0)),
        ],
        out_shape=[
            jax.ShapeDtypeStruct((N, HC), jnp.float32),
            jax.ShapeDtypeStruct((E, NF), jnp.float32),
        ],
    )(rel_pos, edge_attr, z.reshape(-1, 1), tag.reshape(-1, 1), table,
      W_e1, b_e1.reshape(1, -1), W_e12, b_e12.reshape(1, -1),
      W_e2, b_e2.reshape(1, -1),
      W_lin, b_lin.reshape(1, -1), W_lin2, b_lin2.reshape(1, -1))

    return (h, e)


# speedup vs baseline: 1.3023x; 1.0405x over previous
import jax
import jax.numpy as jnp
from jax import lax
from jax.experimental import pallas as pl
from jax.experimental.pallas import tpu as pltpu

N = 50000
E = 800000
NG = 50
NF = 128
HC = 256
TH = 32
EMB_DIM = HC - TH
NZ = 85
NTAG = 3
NROWS = NZ + NTAG

EDGE_BLK = 10000
NSTEPS = E // EDGE_BLK
ATOM_BLK = 1024
ATOM_STEPS = pl.cdiv(N, ATOM_BLK)


def _silu(x):
    return x * (1.0 / (1.0 + jnp.exp(-x)))


def _body(rp_ref, ea_ref, pk_ref, table_ref,
          we1_ref, be1_ref, we12_ref, be12_ref, we2_ref, be2_ref,
          wl_ref, bl_ref, wl2_ref, bl2_ref,
          h_ref, e_ref):
    rp = rp_ref[...]
    ea = ea_ref[...]
    u = _silu(jnp.dot(rp, we1_ref[...], preferred_element_type=jnp.float32)
              + be1_ref[...])
    v = _silu(jnp.dot(ea, we12_ref[...], preferred_element_type=jnp.float32)
              + be12_ref[...])
    pre = (jnp.dot(u, we2_ref[:NF // 2, :], preferred_element_type=jnp.float32)
           + jnp.dot(v, we2_ref[NF // 2:, :], preferred_element_type=jnp.float32)
           + be2_ref[...])
    e_ref[...] = _silu(pre)

    pk = pk_ref[...].reshape(ATOM_BLK, 1)
    zb = jax.lax.shift_right_logical(pk, 2)
    tb = pk & 3
    cols = lax.broadcasted_iota(jnp.int32, (ATOM_BLK, NROWS), 1)
    oh = ((zb == cols) | ((tb + NZ) == cols)).astype(jnp.float32)
    h0 = jnp.dot(oh, table_ref[...], preferred_element_type=jnp.float32)
    h1 = _silu(jnp.dot(h0, wl_ref[...], preferred_element_type=jnp.float32)
               + bl_ref[...])
    h_ref[...] = _silu(jnp.dot(h1, wl2_ref[...],
                               preferred_element_type=jnp.float32)
                       + bl2_ref[...])


def _full(shape):
    return pl.BlockSpec(shape, lambda i: (0,) * len(shape))


def _atom_idx(i):
    return (jnp.minimum(i, ATOM_STEPS - 1), 0)


def kernel(z, rel_pos, edge_attr, tag, emb_table, tag_table, W_lin, b_lin,
           W_lin2, b_lin2, W_e1, b_e1, W_e12, b_e12, W_e2, b_e2):
    table = jnp.zeros((NROWS, HC), dtype=jnp.float32)
    table = table.at[:NZ, :EMB_DIM].set(emb_table)
    table = table.at[NZ:, EMB_DIM:].set(tag_table)

    packed = (z.astype(jnp.int32) * 4 + tag.astype(jnp.int32))

    h, e = pl.pallas_call(
        _body,
        grid=(NSTEPS,),
        in_specs=[
            pl.BlockSpec((EDGE_BLK, 3), lambda i: (i, 0)),
            pl.BlockSpec((EDGE_BLK, NG), lambda i: (i, 0)),
            pl.BlockSpec((ATOM_BLK,), lambda i: (jnp.minimum(i, ATOM_STEPS - 1),)),
            _full((NROWS, HC)),
            _full((3, NF // 2)),
            _full((1, NF // 2)),
            _full((NG, NF - NF // 2)),
            _full((1, NF - NF // 2)),
            _full((NF, NF)),
            _full((1, NF)),
            _full((HC, HC)),
            _full((1, HC)),
            _full((HC, HC)),
            _full((1, HC)),
        ],
        out_specs=[
            pl.BlockSpec((ATOM_BLK, HC), _atom_idx),
            pl.BlockSpec((EDGE_BLK, NF), lambda i: (i, 0)),
        ],
        out_shape=[
            jax.ShapeDtypeStruct((N, HC), jnp.float32),
            jax.ShapeDtypeStruct((E, NF), jnp.float32),
        ],
    )(rel_pos, edge_attr, packed, table,
      W_e1, b_e1.reshape(1, -1), W_e12, b_e12.reshape(1, -1),
      W_e2, b_e2.reshape(1, -1),
      W_lin, b_lin.reshape(1, -1), W_lin2, b_lin2.reshape(1, -1))

    return (h, e)


# speedup vs baseline: 1.3274x; 1.0192x over previous
import jax
import jax.numpy as jnp
from jax import lax
from jax.experimental import pallas as pl
from jax.experimental.pallas import tpu as pltpu

N = 50000
E = 800000
NG = 50
NF = 128
HC = 256
TH = 32
EMB_DIM = HC - TH
NZ = 85
NTAG = 3
NROWS = NZ + NTAG

EDGE_BLK = 12800
NSTEPS = pl.cdiv(E, EDGE_BLK)
ATOM_BLK = 1024
ATOM_STEPS = pl.cdiv(N, ATOM_BLK)


def _silu(x):
    return x * (1.0 / (1.0 + jnp.exp(-x)))


def _body(rp_ref, ea_ref, pk_ref, table_ref,
          we1_ref, be1_ref, we12_ref, be12_ref, we2_ref, be2_ref,
          wl_ref, bl_ref, wl2_ref, bl2_ref,
          h_ref, e_ref):
    rp = rp_ref[...]
    ea = ea_ref[...]
    u = _silu(jnp.dot(rp, we1_ref[...], preferred_element_type=jnp.float32)
              + be1_ref[...])
    v = _silu(jnp.dot(ea, we12_ref[...], preferred_element_type=jnp.float32)
              + be12_ref[...])
    pre = (jnp.dot(u, we2_ref[:NF // 2, :], preferred_element_type=jnp.float32)
           + jnp.dot(v, we2_ref[NF // 2:, :], preferred_element_type=jnp.float32)
           + be2_ref[...])
    e_ref[...] = _silu(pre)

    pk = pk_ref[...].reshape(ATOM_BLK, 1)
    zb = jax.lax.shift_right_logical(pk, 2)
    tb = pk & 3
    cols = lax.broadcasted_iota(jnp.int32, (ATOM_BLK, NROWS), 1)
    oh = ((zb == cols) | ((tb + NZ) == cols)).astype(jnp.float32)
    h0 = jnp.dot(oh, table_ref[...], preferred_element_type=jnp.float32)
    h1 = _silu(jnp.dot(h0, wl_ref[...], preferred_element_type=jnp.float32)
               + bl_ref[...])
    h_ref[...] = _silu(jnp.dot(h1, wl2_ref[...],
                               preferred_element_type=jnp.float32)
                       + bl2_ref[...])


def _full(shape):
    return pl.BlockSpec(shape, lambda i: (0,) * len(shape))


def _atom_idx(i):
    return (jnp.minimum(i, ATOM_STEPS - 1), 0)


def kernel(z, rel_pos, edge_attr, tag, emb_table, tag_table, W_lin, b_lin,
           W_lin2, b_lin2, W_e1, b_e1, W_e12, b_e12, W_e2, b_e2):
    table = jnp.zeros((NROWS, HC), dtype=jnp.float32)
    table = table.at[:NZ, :EMB_DIM].set(emb_table)
    table = table.at[NZ:, EMB_DIM:].set(tag_table)

    packed = (z.astype(jnp.int32) * 4 + tag.astype(jnp.int32))

    h, e = pl.pallas_call(
        _body,
        grid=(NSTEPS,),
        in_specs=[
            pl.BlockSpec((EDGE_BLK, 3), lambda i: (i, 0)),
            pl.BlockSpec((EDGE_BLK, NG), lambda i: (i, 0)),
            pl.BlockSpec((ATOM_BLK,), lambda i: (jnp.minimum(i, ATOM_STEPS - 1),)),
            _full((NROWS, HC)),
            _full((3, NF // 2)),
            _full((1, NF // 2)),
            _full((NG, NF - NF // 2)),
            _full((1, NF - NF // 2)),
            _full((NF, NF)),
            _full((1, NF)),
            _full((HC, HC)),
            _full((1, HC)),
            _full((HC, HC)),
            _full((1, HC)),
        ],
        out_specs=[
            pl.BlockSpec((ATOM_BLK, HC), _atom_idx),
            pl.BlockSpec((EDGE_BLK, NF), lambda i: (i, 0)),
        ],
        out_shape=[
            jax.ShapeDtypeStruct((N, HC), jnp.float32),
            jax.ShapeDtypeStruct((E, NF), jnp.float32),
        ],
    )(rel_pos, edge_attr, packed, table,
      W_e1, b_e1.reshape(1, -1), W_e12, b_e12.reshape(1, -1),
      W_e2, b_e2.reshape(1, -1),
      W_lin, b_lin.reshape(1, -1), W_lin2, b_lin2.reshape(1, -1))

    return (h, e)
